# baseline (speedup 1.0000x reference)
import functools

import jax
import jax.numpy as jnp
from jax import lax
from jax.experimental import pallas as pl
from jax.experimental.pallas import tpu as pltpu
from jax.experimental.pallas import tpu_sc as plsc

N = 10000
E = 320000
D_IN = 128
H = 16
D_OUT = 300

NC = 2
NS = 16
NW = NC * NS
CHUNK = 128
CHUNKS = 80
E_TILE = CHUNKS * CHUNK
E_PAD = NW * E_TILE
N_PAD = 10240
RPT = N_PAD // NS
D_OUT_PAD = 384
BM = 1024

_mesh = plsc.VectorSubcoreMesh(
    core_axis_name="c", subcore_axis_name="s", num_cores=NC, num_subcores=NS)


def _sc_agg(table, src3, dst3, zeros):
    @functools.partial(
        pl.kernel,
        mesh=_mesh,
        out_type=[jax.ShapeDtypeStruct((NC, N_PAD, 128), jnp.float32),
                  jax.ShapeDtypeStruct((N_PAD, H), jnp.float32)],
        compiler_params=pltpu.CompilerParams(use_tc_tiling_on_sc=False),
        scratch_types=[
            pltpu.VMEM((CHUNKS, CHUNK), jnp.int32),
            pltpu.VMEM((CHUNKS, CHUNK), jnp.int32),
            pltpu.VMEM((CHUNK, H), jnp.float32),
            pltpu.VMEM((RPT, H), jnp.float32),
            pltpu.VMEM_SHARED((N_PAD, H), jnp.float32),
            pltpu.SemaphoreType.DMA,
        ],
    )
    def body(table_h, src_h, dst_h, zeros_h, out_h, ctab_h,
             src_v, dst_v, buf, rows_v, acc, sem):
        cid = lax.axis_index("c")
        sid = lax.axis_index("s")
        wid = cid * NS + sid
        row0 = sid * RPT
        pltpu.sync_copy(table_h.at[pl.ds(row0, RPT), pl.ds(0, H)], rows_v)
        pltpu.sync_copy(rows_v, ctab_h.at[pl.ds(row0, RPT)])
        pltpu.sync_copy(zeros_h.at[pl.ds(row0, RPT)], acc.at[pl.ds(row0, RPT)])
        pltpu.sync_copy(src_h.at[wid], src_v)
        pltpu.sync_copy(dst_h.at[wid], dst_v)
        plsc.subcore_barrier()

        def chunk(j, carry):
            pltpu.async_copy(ctab_h.at[src_v.at[j]], buf, sem).wait()
            pltpu.sync_copy(buf, acc.at[dst_v.at[j]], add=True)
            return carry

        lax.fori_loop(0, CHUNKS, chunk, 0)
        plsc.subcore_barrier()
        pltpu.sync_copy(acc.at[pl.ds(row0, RPT)],
                        out_h.at[cid, pl.ds(row0, RPT), pl.ds(0, H)])

    return body(table, src3, dst3, zeros)[0]


def _sc_deg(dst3, e0, zeros):
    @functools.partial(
        pl.kernel,
        mesh=_mesh,
        out_type=jax.ShapeDtypeStruct((NC, N_PAD, 128), jnp.float32),
        compiler_params=pltpu.CompilerParams(use_tc_tiling_on_sc=False),
        scratch_types=[
            pltpu.VMEM((CHUNKS, CHUNK), jnp.int32),
            pltpu.VMEM((CHUNK, H), jnp.float32),
            pltpu.VMEM_SHARED((N_PAD, H), jnp.float32),
            pltpu.SemaphoreType.DMA,
        ],
    )
    def body(dst_h, e0_h, zeros_h, out_h, dst_v, buf, acc, sem):
        cid = lax.axis_index("c")
        sid = lax.axis_index("s")
        wid = cid * NS + sid
        row0 = sid * RPT
        pltpu.sync_copy(zeros_h.at[pl.ds(row0, RPT)], acc.at[pl.ds(row0, RPT)])
        pltpu.sync_copy(dst_h.at[wid], dst_v)
        pltpu.sync_copy(e0_h, buf)
        plsc.subcore_barrier()

        def chunk(j, carry):
            pltpu.sync_copy(buf, acc.at[dst_v.at[j]], add=True)
            return carry

        lax.fori_loop(0, CHUNKS, chunk, 0)
        plsc.subcore_barrier()
        pltpu.sync_copy(acc.at[pl.ds(row0, RPT)],
                        out_h.at[cid, pl.ds(row0, RPT), pl.ds(0, H)])

    return body(dst3, e0, zeros)


def _mm_body(x_ref, w_ref, o_ref):
    h = jnp.dot(x_ref[...], w_ref[...], preferred_element_type=jnp.float32)
    o_ref[:, 0:H] = h


def _tc_matmul(x_p, w0):
    return pl.pallas_call(
        _mm_body,
        grid=(N_PAD // BM,),
        in_specs=[pl.BlockSpec((BM, D_IN), lambda i: (i, 0)),
                  pl.BlockSpec((D_IN, H), lambda i: (0, 0))],
        out_specs=pl.BlockSpec((BM, 128), lambda i: (i, 0)),
        out_shape=jax.ShapeDtypeStruct((N_PAD, 128), jnp.float32),
    )(x_p, w0)


def _scale_body(p0_ref, p1_ref, h0_ref, s_ref, dv_ref):
    deg = p0_ref[:, 0:1] + p1_ref[:, 0:1] + 1.0
    dv = lax.rsqrt(deg)
    s_ref[:, 0:H] = h0_ref[:, 0:H] * dv
    dv_ref[...] = jnp.broadcast_to(dv, (BM, H))


def _tc_scale(p0, p1, h0):
    wide = pl.BlockSpec((BM, 128), lambda i: (i, 0))
    return pl.pallas_call(
        _scale_body,
        grid=(N_PAD // BM,),
        in_specs=[wide, wide, wide],
        out_specs=[wide, pl.BlockSpec((BM, H), lambda i: (i, 0))],
        out_shape=[jax.ShapeDtypeStruct((N_PAD, 128), jnp.float32),
                   jax.ShapeDtypeStruct((N_PAD, H), jnp.float32)],
    )(p0, p1, h0)


def _bn_body(q0_ref, q1_ref, s_ref, dv_ref, b0_ref, mean_ref, var_ref,
             gamma_ref, beta_ref, h_ref, t_ref):
    dv = dv_ref[...]
    scale = gamma_ref[...] * lax.rsqrt(var_ref[...] + 1e-5)
    agg = q0_ref[:, 0:H] + q1_ref[:, 0:H] + s_ref[:, 0:H]
    conv = agg * dv + b0_ref[...]
    h = (conv - mean_ref[...]) * scale + beta_ref[...]
    h_ref[...] = h
    t_ref[:, 0:H] = h * dv


def _tc_bn(q0, q1, s, dv, b0r, meanr, varr, gammar, betar):
    wide = pl.BlockSpec((BM, 128), lambda i: (i, 0))
    nar = pl.BlockSpec((BM, H), lambda i: (i, 0))
    par = pl.BlockSpec((1, H), lambda i: (0, 0))
    return pl.pallas_call(
        _bn_body,
        grid=(N_PAD // BM,),
        in_specs=[wide, wide, wide, nar, par, par, par, par, par],
        out_specs=[nar, wide],
        out_shape=[jax.ShapeDtypeStruct((N_PAD, H), jnp.float32),
                   jax.ShapeDtypeStruct((N_PAD, 128), jnp.float32)],
    )(q0, q1, s, dv, b0r, meanr, varr, gammar, betar)


def _out_body(r0_ref, r1_ref, t_ref, dv_ref, w_ref, b_ref, o_ref):
    agg = r0_ref[:, 0:H] + r1_ref[:, 0:H] + t_ref[:, 0:H]
    u = agg * dv_ref[...]
    emb = jnp.dot(u, w_ref[...], preferred_element_type=jnp.float32) + b_ref[...]
    m = jnp.max(emb, axis=1, keepdims=True)
    z = emb - m
    lse = jnp.log(jnp.sum(jnp.exp(z), axis=1, keepdims=True))
    o_ref[...] = z - lse


def _tc_out(r0, r1, t, dv, w_pad, b_pad):
    wide = pl.BlockSpec((BM, 128), lambda i: (i, 0))
    return pl.pallas_call(
        _out_body,
        grid=(N_PAD // BM,),
        in_specs=[wide, wide, wide,
                  pl.BlockSpec((BM, H), lambda i: (i, 0)),
                  pl.BlockSpec((H, D_OUT_PAD), lambda i: (0, 0)),
                  pl.BlockSpec((1, D_OUT_PAD), lambda i: (0, 0))],
        out_specs=pl.BlockSpec((BM, D_OUT_PAD), lambda i: (i, 0)),
        out_shape=jax.ShapeDtypeStruct((N_PAD, D_OUT_PAD), jnp.float32),
    )(r0, r1, t, dv, w_pad, b_pad)


def kernel(x, edge_index, W0, b0, bn_gamma, bn_beta, bn_mean, bn_var,
           W_out, b_out):
    src = edge_index[0].astype(jnp.int32)
    dst = edge_index[1].astype(jnp.int32)
    idx_pad = jnp.full((E_PAD - E,), N, dtype=jnp.int32)
    src3 = jnp.concatenate([src, idx_pad]).reshape(NW, CHUNKS, CHUNK)
    dst3 = jnp.concatenate([dst, idx_pad]).reshape(NW, CHUNKS, CHUNK)

    x_p = jnp.pad(x, ((0, N_PAD - N), (0, 0)))
    zeros = jnp.zeros((N_PAD, H), jnp.float32)
    e0 = jnp.zeros((CHUNK, H), jnp.float32).at[:, 0].set(1.0)
    w_pad = jnp.pad(W_out, ((0, 0), (0, D_OUT_PAD - D_OUT)))
    b_pad = jnp.concatenate(
        [b_out, jnp.full((D_OUT_PAD - D_OUT,), -1e30, jnp.float32)]
    ).reshape(1, D_OUT_PAD)
    b0r = b0.reshape(1, H)
    meanr = bn_mean.reshape(1, H)
    varr = bn_var.reshape(1, H)
    gammar = bn_gamma.reshape(1, H)
    betar = bn_beta.reshape(1, H)

    h0 = _tc_matmul(x_p, W0)
    degp = _sc_deg(dst3, e0, zeros)
    s, dv = _tc_scale(degp[0], degp[1], h0)
    aggp = _sc_agg(s, src3, dst3, zeros)
    h, t = _tc_bn(aggp[0], aggp[1], s, dv,
                  b0r, meanr, varr, gammar, betar)
    agg2 = _sc_agg(t, src3, dst3, zeros)
    y = _tc_out(agg2[0], agg2[1], t, dv, w_pad, b_pad)

    return (h[:N], y[:N, :D_OUT])


# speedup vs baseline: 1.1569x; 1.1569x over previous
import functools

import jax
import jax.numpy as jnp
from jax import lax
from jax.experimental import pallas as pl
from jax.experimental.pallas import tpu as pltpu
from jax.experimental.pallas import tpu_sc as plsc

N = 10000
E = 320000
D_IN = 128
H = 16
D_OUT = 300

NC = 2
NS = 16
NW = NC * NS
E_TILE = E // NW
CHUNK = 128
CHUNKS = E_TILE // CHUNK
TAIL = E_TILE - CHUNKS * CHUNK
N_PAD = 10240
RPT = N_PAD // NS
D_OUT_PAD = 384
BM = 1024

_mesh = plsc.VectorSubcoreMesh(
    core_axis_name="c", subcore_axis_name="s", num_cores=NC, num_subcores=NS)


def _sc_agg(table, ei, zeros):
    @functools.partial(
        pl.kernel,
        mesh=_mesh,
        out_type=jax.ShapeDtypeStruct((NC, N_PAD, H), jnp.float32),
        compiler_params=pltpu.CompilerParams(use_tc_tiling_on_sc=False),
        scratch_types=[
            pltpu.VMEM((E_TILE,), jnp.int32),
            pltpu.VMEM((E_TILE,), jnp.int32),
            pltpu.VMEM((CHUNK, H), jnp.float32),
            pltpu.VMEM_SHARED((N_PAD, H), jnp.float32),
            pltpu.SemaphoreType.DMA,
        ],
    )
    def body(table_h, ei_h, zeros_h, out_h, src_v, dst_v, buf, acc, sem):
        cid = lax.axis_index("c")
        sid = lax.axis_index("s")
        wid = cid * NS + sid
        row0 = sid * RPT
        e0 = wid * E_TILE
        pltpu.sync_copy(zeros_h.at[pl.ds(row0, RPT)], acc.at[pl.ds(row0, RPT)])
        pltpu.sync_copy(ei_h.at[0, pl.ds(e0, E_TILE)], src_v)
        pltpu.sync_copy(ei_h.at[1, pl.ds(e0, E_TILE)], dst_v)
        plsc.subcore_barrier()

        def chunk(j, carry):
            pltpu.async_copy(
                table_h.at[src_v.at[pl.ds(j * CHUNK, CHUNK)]], buf, sem).wait()
            pltpu.sync_copy(buf, acc.at[dst_v.at[pl.ds(j * CHUNK, CHUNK)]],
                            add=True)
            return carry

        lax.fori_loop(0, CHUNKS, chunk, 0)
        tb = CHUNKS * CHUNK
        pltpu.async_copy(
            table_h.at[src_v.at[pl.ds(tb, TAIL)]], buf.at[pl.ds(0, TAIL)],
            sem).wait()
        pltpu.sync_copy(buf.at[pl.ds(0, TAIL)],
                        acc.at[dst_v.at[pl.ds(tb, TAIL)]], add=True)
        plsc.subcore_barrier()
        pltpu.sync_copy(acc.at[pl.ds(row0, RPT)], out_h.at[cid, pl.ds(row0, RPT)])

    return body(table, ei, zeros)


def _sc_deg(ei, e0c, zeros):
    @functools.partial(
        pl.kernel,
        mesh=_mesh,
        out_type=jax.ShapeDtypeStruct((NC, N_PAD, H), jnp.float32),
        compiler_params=pltpu.CompilerParams(use_tc_tiling_on_sc=False),
        scratch_types=[
            pltpu.VMEM((E_TILE,), jnp.int32),
            pltpu.VMEM((CHUNK, H), jnp.float32),
            pltpu.VMEM_SHARED((N_PAD, H), jnp.float32),
            pltpu.SemaphoreType.DMA,
        ],
    )
    def body(ei_h, e0_h, zeros_h, out_h, dst_v, buf, acc, sem):
        cid = lax.axis_index("c")
        sid = lax.axis_index("s")
        wid = cid * NS + sid
        row0 = sid * RPT
        e0 = wid * E_TILE
        pltpu.sync_copy(zeros_h.at[pl.ds(row0, RPT)], acc.at[pl.ds(row0, RPT)])
        pltpu.sync_copy(ei_h.at[1, pl.ds(e0, E_TILE)], dst_v)
        pltpu.sync_copy(e0_h, buf)
        plsc.subcore_barrier()

        def chunk(j, carry):
            pltpu.sync_copy(buf, acc.at[dst_v.at[pl.ds(j * CHUNK, CHUNK)]],
                            add=True)
            return carry

        lax.fori_loop(0, CHUNKS, chunk, 0)
        tb = CHUNKS * CHUNK
        pltpu.sync_copy(buf.at[pl.ds(0, TAIL)],
                        acc.at[dst_v.at[pl.ds(tb, TAIL)]], add=True)
        plsc.subcore_barrier()
        pltpu.sync_copy(acc.at[pl.ds(row0, RPT)], out_h.at[cid, pl.ds(row0, RPT)])

    return body(ei, e0c, zeros)


def _mm_body(x_ref, w_ref, o_ref):
    o_ref[...] = jnp.dot(x_ref[...], w_ref[...],
                         preferred_element_type=jnp.float32)


def _tc_matmul(x_p, w0):
    return pl.pallas_call(
        _mm_body,
        grid=(N_PAD // BM,),
        in_specs=[pl.BlockSpec((BM, D_IN), lambda i: (i, 0)),
                  pl.BlockSpec((D_IN, H), lambda i: (0, 0))],
        out_specs=pl.BlockSpec((BM, H), lambda i: (i, 0)),
        out_shape=jax.ShapeDtypeStruct((N_PAD, H), jnp.float32),
    )(x_p, w0)


def _scale_body(p0_ref, p1_ref, h0_ref, s_ref, dv_ref):
    deg = p0_ref[:, 0:1] + p1_ref[:, 0:1] + 1.0
    dv = lax.rsqrt(deg)
    s_ref[...] = h0_ref[...] * dv
    dv_ref[...] = jnp.broadcast_to(dv, (BM, H))


def _tc_scale(p0, p1, h0):
    blk = pl.BlockSpec((BM, H), lambda i: (i, 0))
    return pl.pallas_call(
        _scale_body,
        grid=(N_PAD // BM,),
        in_specs=[blk, blk, blk],
        out_specs=[blk, blk],
        out_shape=[jax.ShapeDtypeStruct((N_PAD, H), jnp.float32),
                   jax.ShapeDtypeStruct((N_PAD, H), jnp.float32)],
    )(p0, p1, h0)


def _bn_body(q0_ref, q1_ref, s_ref, dv_ref, b0_ref, mean_ref, var_ref,
             gamma_ref, beta_ref, h_ref, t_ref):
    dv = dv_ref[...]
    scale = gamma_ref[...] * lax.rsqrt(var_ref[...] + 1e-5)
    conv = (q0_ref[...] + q1_ref[...] + s_ref[...]) * dv + b0_ref[...]
    h = (conv - mean_ref[...]) * scale + beta_ref[...]
    h_ref[...] = h
    t_ref[...] = h * dv


def _tc_bn(q0, q1, s, dv, b0r, meanr, varr, gammar, betar):
    blk = pl.BlockSpec((BM, H), lambda i: (i, 0))
    par = pl.BlockSpec((1, H), lambda i: (0, 0))
    return pl.pallas_call(
        _bn_body,
        grid=(N_PAD // BM,),
        in_specs=[blk, blk, blk, blk, par, par, par, par, par],
        out_specs=[blk, blk],
        out_shape=[jax.ShapeDtypeStruct((N_PAD, H), jnp.float32),
                   jax.ShapeDtypeStruct((N_PAD, H), jnp.float32)],
    )(q0, q1, s, dv, b0r, meanr, varr, gammar, betar)


def _out_body(r0_ref, r1_ref, t_ref, dv_ref, w_ref, b_ref, o_ref):
    u = (r0_ref[...] + r1_ref[...] + t_ref[...]) * dv_ref[...]
    emb = jnp.dot(u, w_ref[...], preferred_element_type=jnp.float32) + b_ref[...]
    m = jnp.max(emb, axis=1, keepdims=True)
    z = emb - m
    lse = jnp.log(jnp.sum(jnp.exp(z), axis=1, keepdims=True))
    o_ref[...] = z - lse


def _tc_out(r0, r1, t, dv, w_pad, b_pad):
    blk = pl.BlockSpec((BM, H), lambda i: (i, 0))
    return pl.pallas_call(
        _out_body,
        grid=(N_PAD // BM,),
        in_specs=[blk, blk, blk, blk,
                  pl.BlockSpec((H, D_OUT_PAD), lambda i: (0, 0)),
                  pl.BlockSpec((1, D_OUT_PAD), lambda i: (0, 0))],
        out_specs=pl.BlockSpec((BM, D_OUT_PAD), lambda i: (i, 0)),
        out_shape=jax.ShapeDtypeStruct((N_PAD, D_OUT_PAD), jnp.float32),
    )(r0, r1, t, dv, w_pad, b_pad)


def kernel(x, edge_index, W0, b0, bn_gamma, bn_beta, bn_mean, bn_var,
           W_out, b_out):
    ei = edge_index.astype(jnp.int32)

    x_p = jnp.pad(x, ((0, N_PAD - N), (0, 0)))
    zeros = jnp.zeros((N_PAD, H), jnp.float32)
    e0c = jnp.zeros((CHUNK, H), jnp.float32).at[:, 0].set(1.0)
    w_pad = jnp.pad(W_out, ((0, 0), (0, D_OUT_PAD - D_OUT)))
    b_pad = jnp.concatenate(
        [b_out, jnp.full((D_OUT_PAD - D_OUT,), -1e30, jnp.float32)]
    ).reshape(1, D_OUT_PAD)
    b0r = b0.reshape(1, H)
    meanr = bn_mean.reshape(1, H)
    varr = bn_var.reshape(1, H)
    gammar = bn_gamma.reshape(1, H)
    betar = bn_beta.reshape(1, H)

    h0 = _tc_matmul(x_p, W0)
    degp = _sc_deg(ei, e0c, zeros)
    s, dv = _tc_scale(degp[0], degp[1], h0)
    aggp = _sc_agg(s, ei, zeros)
    h, t = _tc_bn(aggp[0], aggp[1], s, dv,
                  b0r, meanr, varr, gammar, betar)
    agg2 = _sc_agg(t, ei, zeros)
    y = _tc_out(agg2[0], agg2[1], t, dv, w_pad, b_pad)

    return (h[:N], y[:N, :D_OUT])


# speedup vs baseline: 1.4281x; 1.2344x over previous
import functools

import jax
import jax.numpy as jnp
from jax import lax
from jax.experimental import pallas as pl
from jax.experimental.pallas import tpu as pltpu
from jax.experimental.pallas import tpu_sc as plsc

N = 10000
E = 320000
D_IN = 128
H = 16
D_OUT = 300

NC = 2
NS = 16
NW = NC * NS
E_TILE = E // NW
CHUNK = 128
CHUNKS = E_TILE // CHUNK
TAIL = E_TILE - CHUNKS * CHUNK
N_PAD = 10240
RPT = N_PAD // NS
D_OUT_PAD = 304
BM = 1024
BT = 2000

_mesh = plsc.VectorSubcoreMesh(
    core_axis_name="c", subcore_axis_name="s", num_cores=NC, num_subcores=NS)


def _sc_agg(table, ei, zeros):
    @functools.partial(
        pl.kernel,
        mesh=_mesh,
        out_type=jax.ShapeDtypeStruct((NC, N_PAD, H), jnp.float32),
        compiler_params=pltpu.CompilerParams(use_tc_tiling_on_sc=False),
        scratch_types=[
            pltpu.VMEM((E_TILE,), jnp.int32),
            pltpu.VMEM((E_TILE,), jnp.int32),
            pltpu.VMEM((CHUNK, H), jnp.float32),
            pltpu.VMEM_SHARED((N_PAD, H), jnp.float32),
            pltpu.SemaphoreType.DMA,
        ],
    )
    def body(table_h, ei_h, zeros_h, out_h, src_v, dst_v, buf, acc, sem):
        cid = lax.axis_index("c")
        sid = lax.axis_index("s")
        wid = cid * NS + sid
        row0 = sid * RPT
        e0 = wid * E_TILE
        pltpu.sync_copy(zeros_h.at[pl.ds(row0, RPT)], acc.at[pl.ds(row0, RPT)])
        pltpu.sync_copy(ei_h.at[0, pl.ds(e0, E_TILE)], src_v)
        pltpu.sync_copy(ei_h.at[1, pl.ds(e0, E_TILE)], dst_v)
        plsc.subcore_barrier()

        def chunk(j, carry):
            pltpu.async_copy(
                table_h.at[src_v.at[pl.ds(j * CHUNK, CHUNK)]], buf, sem).wait()
            pltpu.sync_copy(buf, acc.at[dst_v.at[pl.ds(j * CHUNK, CHUNK)]],
                            add=True)
            return carry

        lax.fori_loop(0, CHUNKS, chunk, 0)
        tb = CHUNKS * CHUNK
        pltpu.async_copy(
            table_h.at[src_v.at[pl.ds(tb, TAIL)]], buf.at[pl.ds(0, TAIL)],
            sem).wait()
        pltpu.sync_copy(buf.at[pl.ds(0, TAIL)],
                        acc.at[dst_v.at[pl.ds(tb, TAIL)]], add=True)
        plsc.subcore_barrier()
        pltpu.sync_copy(acc.at[pl.ds(row0, RPT)], out_h.at[cid, pl.ds(row0, RPT)])

    return body(table, ei, zeros)


def _sc_deg(ei, e0c, zeros):
    @functools.partial(
        pl.kernel,
        mesh=_mesh,
        out_type=jax.ShapeDtypeStruct((NC, N_PAD, H), jnp.float32),
        compiler_params=pltpu.CompilerParams(use_tc_tiling_on_sc=False),
        scratch_types=[
            pltpu.VMEM((E_TILE,), jnp.int32),
            pltpu.VMEM((CHUNK, H), jnp.float32),
            pltpu.VMEM_SHARED((N_PAD, H), jnp.float32),
            pltpu.SemaphoreType.DMA,
        ],
    )
    def body(ei_h, e0_h, zeros_h, out_h, dst_v, buf, acc, sem):
        cid = lax.axis_index("c")
        sid = lax.axis_index("s")
        wid = cid * NS + sid
        row0 = sid * RPT
        e0 = wid * E_TILE
        pltpu.sync_copy(zeros_h.at[pl.ds(row0, RPT)], acc.at[pl.ds(row0, RPT)])
        pltpu.sync_copy(ei_h.at[1, pl.ds(e0, E_TILE)], dst_v)
        pltpu.sync_copy(e0_h, buf)
        plsc.subcore_barrier()

        def chunk(j, carry):
            pltpu.sync_copy(buf, acc.at[dst_v.at[pl.ds(j * CHUNK, CHUNK)]],
                            add=True)
            return carry

        lax.fori_loop(0, CHUNKS, chunk, 0)
        tb = CHUNKS * CHUNK
        pltpu.sync_copy(buf.at[pl.ds(0, TAIL)],
                        acc.at[dst_v.at[pl.ds(tb, TAIL)]], add=True)
        plsc.subcore_barrier()
        pltpu.sync_copy(acc.at[pl.ds(row0, RPT)], out_h.at[cid, pl.ds(row0, RPT)])

    return body(ei, e0c, zeros)


def _mm_body(x_ref, w_ref, o_ref):
    o_ref[...] = jnp.dot(x_ref[...], w_ref[...],
                         preferred_element_type=jnp.float32)


def _tc_matmul(x_p, w0):
    return pl.pallas_call(
        _mm_body,
        grid=(N_PAD // BM,),
        in_specs=[pl.BlockSpec((BM, D_IN), lambda i: (i, 0)),
                  pl.BlockSpec((D_IN, H), lambda i: (0, 0))],
        out_specs=pl.BlockSpec((BM, H), lambda i: (i, 0)),
        out_shape=jax.ShapeDtypeStruct((N_PAD, H), jnp.float32),
    )(x_p, w0)


def _scale_body(p0_ref, p1_ref, h0_ref, s_ref, dv_ref):
    deg = p0_ref[:, 0:1] + p1_ref[:, 0:1] + 1.0
    dv = lax.rsqrt(deg)
    s_ref[...] = h0_ref[...] * dv
    dv_ref[...] = jnp.broadcast_to(dv, (BM, H))


def _tc_scale(p0, p1, h0):
    blk = pl.BlockSpec((BM, H), lambda i: (i, 0))
    return pl.pallas_call(
        _scale_body,
        grid=(N_PAD // BM,),
        in_specs=[blk, blk, blk],
        out_specs=[blk, blk],
        out_shape=[jax.ShapeDtypeStruct((N_PAD, H), jnp.float32),
                   jax.ShapeDtypeStruct((N_PAD, H), jnp.float32)],
    )(p0, p1, h0)


def _bn_body(q0_ref, q1_ref, s_ref, dv_ref, b0_ref, mean_ref, var_ref,
             gamma_ref, beta_ref, h_ref, t_ref):
    dv = dv_ref[...]
    scale = gamma_ref[...] * lax.rsqrt(var_ref[...] + 1e-5)
    conv = (q0_ref[...] + q1_ref[...] + s_ref[...]) * dv + b0_ref[...]
    h = (conv - mean_ref[...]) * scale + beta_ref[...]
    h_ref[...] = h
    t_ref[...] = h * dv


def _tc_bn(q0, q1, s, dv, b0r, meanr, varr, gammar, betar):
    blk = pl.BlockSpec((BM, H), lambda i: (i, 0))
    par = pl.BlockSpec((1, H), lambda i: (0, 0))
    return pl.pallas_call(
        _bn_body,
        grid=(N_PAD // BM,),
        in_specs=[blk, blk, blk, blk, par, par, par, par, par],
        out_specs=[blk, blk],
        out_shape=[jax.ShapeDtypeStruct((N_PAD, H), jnp.float32),
                   jax.ShapeDtypeStruct((N_PAD, H), jnp.float32)],
    )(q0, q1, s, dv, b0r, meanr, varr, gammar, betar)


def _out_body(r0_ref, r1_ref, t_ref, dv_ref, w_ref, b_ref, o_ref):
    u = (r0_ref[...] + r1_ref[...] + t_ref[...]) * dv_ref[...]
    emb = lax.dot_general(w_ref[...], u, (((0,), (1,)), ((), ())),
                          preferred_element_type=jnp.float32) + b_ref[...]
    m = jnp.max(emb, axis=0, keepdims=True)
    z = emb - m
    lse = jnp.log(jnp.sum(jnp.exp(z), axis=0, keepdims=True))
    o_ref[...] = z - lse


def _tc_out(r0, r1, t, dv, w_pad, b_pad):
    blk = pl.BlockSpec((N, H), lambda i: (0, 0))
    return pl.pallas_call(
        _out_body,
        grid=(1,),
        in_specs=[blk, blk, blk, blk,
                  pl.BlockSpec((H, D_OUT_PAD), lambda i: (0, 0)),
                  pl.BlockSpec((D_OUT_PAD, 1), lambda i: (0, 0))],
        out_specs=pl.BlockSpec((D_OUT_PAD, N), lambda i: (0, 0)),
        out_shape=jax.ShapeDtypeStruct((D_OUT_PAD, N), jnp.float32),
        compiler_params=pltpu.CompilerParams(
            vmem_limit_bytes=48 * 1024 * 1024),
    )(r0, r1, t, dv, w_pad, b_pad)


def kernel(x, edge_index, W0, b0, bn_gamma, bn_beta, bn_mean, bn_var,
           W_out, b_out):
    ei = edge_index.astype(jnp.int32)

    x_p = jnp.pad(x, ((0, N_PAD - N), (0, 0)))
    zeros = jnp.zeros((N_PAD, H), jnp.float32)
    e0c = jnp.zeros((CHUNK, H), jnp.float32).at[:, 0].set(1.0)
    w_pad = jnp.pad(W_out, ((0, 0), (0, D_OUT_PAD - D_OUT)))
    b_pad = jnp.concatenate(
        [b_out, jnp.full((D_OUT_PAD - D_OUT,), -1e30, jnp.float32)]
    ).reshape(D_OUT_PAD, 1)
    b0r = b0.reshape(1, H)
    meanr = bn_mean.reshape(1, H)
    varr = bn_var.reshape(1, H)
    gammar = bn_gamma.reshape(1, H)
    betar = bn_beta.reshape(1, H)

    h0 = _tc_matmul(x_p, W0)
    degp = _sc_deg(ei, e0c, zeros)
    s, dv = _tc_scale(degp[0], degp[1], h0)
    aggp = _sc_agg(s, ei, zeros)
    h, t = _tc_bn(aggp[0], aggp[1], s, dv,
                  b0r, meanr, varr, gammar, betar)
    agg2 = _sc_agg(t, ei, zeros)
    yt = _tc_out(agg2[0], agg2[1], t, dv, w_pad, b_pad)

    return (h[:N], yt[:D_OUT].T)


# speedup vs baseline: 1.8198x; 1.2743x over previous
import functools

import jax
import jax.numpy as jnp
from jax import lax
from jax.experimental import pallas as pl
from jax.experimental.pallas import tpu as pltpu
from jax.experimental.pallas import tpu_sc as plsc

N = 10000
E = 320000
D_IN = 128
H = 16
D_OUT = 300

NC = 2
NS = 16
NW = NC * NS
E_TILE = E // NW
CHUNK = 128
CHUNKS = E_TILE // CHUNK
TAIL = E_TILE - CHUNKS * CHUNK
N_PAD = 10240
RPT = N_PAD // NS
D_OUT_PAD = 304
BM = 1024
BT = 2000

_mesh = plsc.VectorSubcoreMesh(
    core_axis_name="c", subcore_axis_name="s", num_cores=NC, num_subcores=NS)


def _sc_agg(table, ei, zeros):
    @functools.partial(
        pl.kernel,
        mesh=_mesh,
        out_type=jax.ShapeDtypeStruct((NC, N_PAD, H), jnp.float32),
        compiler_params=pltpu.CompilerParams(use_tc_tiling_on_sc=False),
        scratch_types=[
            pltpu.VMEM((E_TILE,), jnp.int32),
            pltpu.VMEM((E_TILE,), jnp.int32),
            pltpu.VMEM((CHUNK, H), jnp.float32),
            pltpu.VMEM((CHUNK, H), jnp.float32),
            pltpu.VMEM_SHARED((N_PAD, H), jnp.float32),
            pltpu.SemaphoreType.DMA,
            pltpu.SemaphoreType.DMA,
        ],
    )
    def body(table_h, ei_h, zeros_h, out_h, src_v, dst_v, buf0, buf1, acc,
             sem0, sem1):
        cid = lax.axis_index("c")
        sid = lax.axis_index("s")
        wid = cid * NS + sid
        row0 = sid * RPT
        e0 = wid * E_TILE
        pltpu.sync_copy(zeros_h.at[pl.ds(row0, RPT)], acc.at[pl.ds(row0, RPT)])
        pltpu.sync_copy(ei_h.at[0, pl.ds(e0, E_TILE)], src_v)
        pltpu.sync_copy(ei_h.at[1, pl.ds(e0, E_TILE)], dst_v)
        plsc.subcore_barrier()

        def gather(j, buf, sem):
            return pltpu.async_copy(
                table_h.at[src_v.at[pl.ds(j * CHUNK, CHUNK)]], buf, sem)

        def gwait(buf, sem):
            pltpu.make_async_copy(table_h.at[pl.ds(0, CHUNK)], buf, sem).wait()

        def scat(j, buf):
            pltpu.sync_copy(buf, acc.at[dst_v.at[pl.ds(j * CHUNK, CHUNK)]],
                            add=True)

        gather(0, buf0, sem0)

        def pair(i, carry):
            j = 2 * i
            gather(j + 1, buf1, sem1)
            gwait(buf0, sem0)
            scat(j, buf0)

            @pl.when(i < CHUNKS // 2 - 1)
            def _():
                gather(j + 2, buf0, sem0)

            gwait(buf1, sem1)
            scat(j + 1, buf1)
            return carry

        lax.fori_loop(0, CHUNKS // 2, pair, 0)
        tb = CHUNKS * CHUNK
        pltpu.async_copy(
            table_h.at[src_v.at[pl.ds(tb, TAIL)]], buf0.at[pl.ds(0, TAIL)],
            sem0).wait()
        pltpu.sync_copy(buf0.at[pl.ds(0, TAIL)],
                        acc.at[dst_v.at[pl.ds(tb, TAIL)]], add=True)
        plsc.subcore_barrier()
        pltpu.sync_copy(acc.at[pl.ds(row0, RPT)], out_h.at[cid, pl.ds(row0, RPT)])

    return body(table, ei, zeros)


def _sc_deg(ei, e0c, zeros):
    @functools.partial(
        pl.kernel,
        mesh=_mesh,
        out_type=jax.ShapeDtypeStruct((NC, N_PAD, H), jnp.float32),
        compiler_params=pltpu.CompilerParams(use_tc_tiling_on_sc=False),
        scratch_types=[
            pltpu.VMEM((E_TILE,), jnp.int32),
            pltpu.VMEM((CHUNK, H), jnp.float32),
            pltpu.VMEM_SHARED((N_PAD, H), jnp.float32),
            pltpu.SemaphoreType.DMA,
        ],
    )
    def body(ei_h, e0_h, zeros_h, out_h, dst_v, buf, acc, sem):
        cid = lax.axis_index("c")
        sid = lax.axis_index("s")
        wid = cid * NS + sid
        row0 = sid * RPT
        e0 = wid * E_TILE
        pltpu.sync_copy(zeros_h.at[pl.ds(row0, RPT)], acc.at[pl.ds(row0, RPT)])
        pltpu.sync_copy(ei_h.at[1, pl.ds(e0, E_TILE)], dst_v)
        pltpu.sync_copy(e0_h, buf)
        plsc.subcore_barrier()

        def chunk(j, carry):
            pltpu.sync_copy(buf, acc.at[dst_v.at[pl.ds(j * CHUNK, CHUNK)]],
                            add=True)
            return carry

        lax.fori_loop(0, CHUNKS, chunk, 0)
        tb = CHUNKS * CHUNK
        pltpu.sync_copy(buf.at[pl.ds(0, TAIL)],
                        acc.at[dst_v.at[pl.ds(tb, TAIL)]], add=True)
        plsc.subcore_barrier()
        pltpu.sync_copy(acc.at[pl.ds(row0, RPT)], out_h.at[cid, pl.ds(row0, RPT)])

    return body(ei, e0c, zeros)


def _mm_body(x_ref, w_ref, o_ref):
    o_ref[...] = jnp.dot(x_ref[...], w_ref[...],
                         preferred_element_type=jnp.float32)


def _tc_matmul(x_p, w0):
    return pl.pallas_call(
        _mm_body,
        grid=(N_PAD // BM,),
        in_specs=[pl.BlockSpec((BM, D_IN), lambda i: (i, 0)),
                  pl.BlockSpec((D_IN, H), lambda i: (0, 0))],
        out_specs=pl.BlockSpec((BM, H), lambda i: (i, 0)),
        out_shape=jax.ShapeDtypeStruct((N_PAD, H), jnp.float32),
    )(x_p, w0)


def _scale_body(p0_ref, p1_ref, h0_ref, s_ref, dv_ref):
    deg = p0_ref[:, 0:1] + p1_ref[:, 0:1] + 1.0
    dv = lax.rsqrt(deg)
    s_ref[...] = h0_ref[...] * dv
    dv_ref[...] = jnp.broadcast_to(dv, (BM, H))


def _tc_scale(p0, p1, h0):
    blk = pl.BlockSpec((BM, H), lambda i: (i, 0))
    return pl.pallas_call(
        _scale_body,
        grid=(N_PAD // BM,),
        in_specs=[blk, blk, blk],
        out_specs=[blk, blk],
        out_shape=[jax.ShapeDtypeStruct((N_PAD, H), jnp.float32),
                   jax.ShapeDtypeStruct((N_PAD, H), jnp.float32)],
    )(p0, p1, h0)


def _bn_body(q0_ref, q1_ref, s_ref, dv_ref, b0_ref, mean_ref, var_ref,
             gamma_ref, beta_ref, h_ref, t_ref):
    dv = dv_ref[...]
    scale = gamma_ref[...] * lax.rsqrt(var_ref[...] + 1e-5)
    conv = (q0_ref[...] + q1_ref[...] + s_ref[...]) * dv + b0_ref[...]
    h = (conv - mean_ref[...]) * scale + beta_ref[...]
    h_ref[...] = h
    t_ref[...] = h * dv


def _tc_bn(q0, q1, s, dv, b0r, meanr, varr, gammar, betar):
    blk = pl.BlockSpec((BM, H), lambda i: (i, 0))
    par = pl.BlockSpec((1, H), lambda i: (0, 0))
    return pl.pallas_call(
        _bn_body,
        grid=(N_PAD // BM,),
        in_specs=[blk, blk, blk, blk, par, par, par, par, par],
        out_specs=[blk, blk],
        out_shape=[jax.ShapeDtypeStruct((N_PAD, H), jnp.float32),
                   jax.ShapeDtypeStruct((N_PAD, H), jnp.float32)],
    )(q0, q1, s, dv, b0r, meanr, varr, gammar, betar)


def _out_body(r0_ref, r1_ref, t_ref, dv_ref, w_ref, b_ref, o_ref):
    u = (r0_ref[...] + r1_ref[...] + t_ref[...]) * dv_ref[...]
    emb = lax.dot_general(w_ref[...], u, (((0,), (1,)), ((), ())),
                          preferred_element_type=jnp.float32) + b_ref[...]
    m = jnp.max(emb, axis=0, keepdims=True)
    z = emb - m
    lse = jnp.log(jnp.sum(jnp.exp(z), axis=0, keepdims=True))
    o_ref[...] = z - lse


def _tc_out(r0, r1, t, dv, w_pad, b_pad):
    blk = pl.BlockSpec((N, H), lambda i: (0, 0))
    return pl.pallas_call(
        _out_body,
        grid=(1,),
        in_specs=[blk, blk, blk, blk,
                  pl.BlockSpec((H, D_OUT_PAD), lambda i: (0, 0)),
                  pl.BlockSpec((D_OUT_PAD, 1), lambda i: (0, 0))],
        out_specs=pl.BlockSpec((D_OUT_PAD, N), lambda i: (0, 0)),
        out_shape=jax.ShapeDtypeStruct((D_OUT_PAD, N), jnp.float32),
        compiler_params=pltpu.CompilerParams(
            vmem_limit_bytes=48 * 1024 * 1024),
    )(r0, r1, t, dv, w_pad, b_pad)


def kernel(x, edge_index, W0, b0, bn_gamma, bn_beta, bn_mean, bn_var,
           W_out, b_out):
    ei = edge_index.astype(jnp.int32)

    x_p = jnp.pad(x, ((0, N_PAD - N), (0, 0)))
    zeros = jnp.zeros((N_PAD, H), jnp.float32)
    e0c = jnp.zeros((CHUNK, H), jnp.float32).at[:, 0].set(1.0)
    w_pad = jnp.pad(W_out, ((0, 0), (0, D_OUT_PAD - D_OUT)))
    b_pad = jnp.concatenate(
        [b_out, jnp.full((D_OUT_PAD - D_OUT,), -1e30, jnp.float32)]
    ).reshape(D_OUT_PAD, 1)
    b0r = b0.reshape(1, H)
    meanr = bn_mean.reshape(1, H)
    varr = bn_var.reshape(1, H)
    gammar = bn_gamma.reshape(1, H)
    betar = bn_beta.reshape(1, H)

    h0 = _tc_matmul(x_p, W0)
    degp = _sc_deg(ei, e0c, zeros)
    s, dv = _tc_scale(degp[0], degp[1], h0)
    aggp = _sc_agg(s, ei, zeros)
    h, t = _tc_bn(aggp[0], aggp[1], s, dv,
                  b0r, meanr, varr, gammar, betar)
    agg2 = _sc_agg(t, ei, zeros)
    yt = _tc_out(agg2[0], agg2[1], t, dv, w_pad, b_pad)

    return (h[:N], yt[:D_OUT].T)
